# Initial kernel scaffold; baseline (speedup 1.0000x reference)
import jax
import jax.numpy as jnp
from jax.experimental import pallas as pl


def kernel(x, edge_index, z_in, z_out):
    raise NotImplementedError("write your pallas kernel here")


# speedup vs baseline: 1.6661x; 1.6661x over previous
import functools

import jax
import jax.numpy as jnp
from jax import lax
from jax.experimental import pallas as pl
from jax.experimental.pallas import tpu as pltpu
from jax.experimental.pallas import tpu_sc as plsc

N_NODES = 10000
NODE_DIM = 128
N_EDGES = 320000
MAX_DEG = 512

NH = 10240
N_CORES = 2
N_SUB = 16
NW = N_CORES * N_SUB
EPT = 10240
E_PAD = EPT * NW
GROUPS = EPT // 128
SLICE = NH // N_SUB


def _hist_body(src_hbm, dst_hbm, out_hbm,
               idx_in_v, idx_out_v, ones_v, zero_v, hist_in_sh, hist_out_sh):
    c = lax.axis_index("c")
    s = lax.axis_index("s")
    g = c * N_SUB + s

    for i in range(SLICE // 16):
        zero_v[pl.ds(i * 16, 16)] = jnp.zeros((16,), jnp.int32)
    for i in range(128 // 16):
        ones_v[pl.ds(i * 16, 16)] = jnp.ones((16,), jnp.int32)
    pltpu.sync_copy(zero_v, hist_in_sh.at[pl.ds(s * SLICE, SLICE)])
    pltpu.sync_copy(zero_v, hist_out_sh.at[pl.ds(s * SLICE, SLICE)])
    plsc.subcore_barrier()

    pltpu.sync_copy(dst_hbm.at[pl.ds(g * GROUPS, GROUPS)], idx_in_v)
    pltpu.sync_copy(src_hbm.at[pl.ds(g * GROUPS, GROUPS)], idx_out_v)

    def body(j, carry):
        pltpu.sync_copy(ones_v, hist_in_sh.at[idx_in_v.at[j]], add=True)
        pltpu.sync_copy(ones_v, hist_out_sh.at[idx_out_v.at[j]], add=True)
        return carry

    lax.fori_loop(0, GROUPS, body, 0)
    plsc.subcore_barrier()

    pltpu.sync_copy(hist_in_sh.at[pl.ds(s * SLICE, SLICE)],
                    out_hbm.at[c, 0, pl.ds(s * SLICE, SLICE)])
    pltpu.sync_copy(hist_out_sh.at[pl.ds(s * SLICE, SLICE)],
                    out_hbm.at[c, 1, pl.ds(s * SLICE, SLICE)])


_hist = functools.partial(
    pl.kernel,
    out_type=jax.ShapeDtypeStruct((N_CORES, 2, NH), jnp.int32),
    mesh=plsc.VectorSubcoreMesh(core_axis_name="c", subcore_axis_name="s"),
    scratch_types=[
        pltpu.VMEM((GROUPS, 128), jnp.int32),
        pltpu.VMEM((GROUPS, 128), jnp.int32),
        pltpu.VMEM((128,), jnp.int32),
        pltpu.VMEM((SLICE,), jnp.int32),
        pltpu.VMEM_SHARED((NH,), jnp.int32),
        pltpu.VMEM_SHARED((NH,), jnp.int32),
    ],
)(_hist_body)


BN = 2000


def _enc_body(h_ref, x_ref, zin_ref, zout_ref, o_ref):
    h = h_ref[...]
    deg_in = jnp.minimum(h[:, 0] + h[:, 1], MAX_DEG - 1)
    deg_out = jnp.minimum(h[:, 2] + h[:, 3], MAX_DEG - 1)
    iota = lax.broadcasted_iota(jnp.int32, (BN, MAX_DEG), 1)
    oh_in = (deg_in[:, None] == iota).astype(jnp.float32)
    oh_out = (deg_out[:, None] == iota).astype(jnp.float32)
    o_ref[...] = (x_ref[...]
                  + jnp.dot(oh_in, zin_ref[...], preferred_element_type=jnp.float32)
                  + jnp.dot(oh_out, zout_ref[...], preferred_element_type=jnp.float32))


def kernel(x, edge_index, z_in, z_out):
    src = edge_index[0].astype(jnp.int32)
    dst = edge_index[1].astype(jnp.int32)
    pad = jnp.full((E_PAD - N_EDGES,), N_NODES + 100, jnp.int32)
    src_g = jnp.concatenate([src, pad]).reshape(E_PAD // 128, 128)
    dst_g = jnp.concatenate([dst, pad]).reshape(E_PAD // 128, 128)

    hist = _hist(src_g, dst_g)
    h4 = hist.transpose(1, 0, 2).reshape(4, NH).T

    return pl.pallas_call(
        _enc_body,
        grid=(N_NODES // BN,),
        in_specs=[
            pl.BlockSpec((BN, 4), lambda i: (i, 0)),
            pl.BlockSpec((BN, NODE_DIM), lambda i: (i, 0)),
            pl.BlockSpec((MAX_DEG, NODE_DIM), lambda i: (0, 0)),
            pl.BlockSpec((MAX_DEG, NODE_DIM), lambda i: (0, 0)),
        ],
        out_specs=pl.BlockSpec((BN, NODE_DIM), lambda i: (i, 0)),
        out_shape=jax.ShapeDtypeStruct((N_NODES, NODE_DIM), jnp.float32),
    )(h4, x, z_in, z_out)


# speedup vs baseline: 2.3439x; 1.4068x over previous
import functools

import jax
import jax.numpy as jnp
from jax import lax
from jax.experimental import pallas as pl
from jax.experimental.pallas import tpu as pltpu
from jax.experimental.pallas import tpu_sc as plsc

N_NODES = 10000
NODE_DIM = 128
N_EDGES = 320000
MAX_DEG = 512

NH = 10240
N_CORES = 2
N_SUB = 16
NW = N_CORES * N_SUB
EPT = 10240
E_PAD = EPT * NW
GROUPS = EPT // 128
SLICE = NH // N_SUB
PAD_ID = N_NODES + 100
K_BURST = 8


def _hist_body(edges_hbm, out_hbm,
               idx_in_v, idx_out_v, ones_v, zero_v,
               hist_in_sh, hist_out_sh, sem):
    c = lax.axis_index("c")
    s = lax.axis_index("s")
    g = c * N_SUB + s

    for i in range(SLICE // 16):
        zero_v[pl.ds(i * 16, 16)] = jnp.zeros((16,), jnp.int32)
    for i in range(128 // 16):
        ones_v[pl.ds(i * 16, 16)] = jnp.ones((16,), jnp.int32)
    pltpu.sync_copy(zero_v, hist_in_sh.at[pl.ds(s * SLICE, SLICE)])
    pltpu.sync_copy(zero_v, hist_out_sh.at[pl.ds(s * SLICE, SLICE)])

    pltpu.sync_copy(edges_hbm.at[1, pl.ds(g * GROUPS, GROUPS)], idx_in_v)
    pltpu.sync_copy(edges_hbm.at[0, pl.ds(g * GROUPS, GROUPS)], idx_out_v)
    plsc.subcore_barrier()

    def body(jj, carry):
        copies = []
        for r in range(K_BURST):
            j = jj * K_BURST + r
            c1 = pltpu.make_async_copy(ones_v, hist_in_sh.at[idx_in_v.at[j]], sem)
            c2 = pltpu.make_async_copy(ones_v, hist_out_sh.at[idx_out_v.at[j]], sem)
            c1.start(add=True)
            c2.start(add=True)
            copies.append(c1)
            copies.append(c2)
        for cp in copies:
            cp.wait()
        return carry

    lax.fori_loop(0, GROUPS // K_BURST, body, 0)
    plsc.subcore_barrier()

    pltpu.sync_copy(hist_in_sh.at[pl.ds(s * SLICE, SLICE)],
                    out_hbm.at[c, pl.ds(s * SLICE, SLICE)])
    pltpu.sync_copy(hist_out_sh.at[pl.ds(s * SLICE, SLICE)],
                    out_hbm.at[2 + c, pl.ds(s * SLICE, SLICE)])


_hist = functools.partial(
    pl.kernel,
    out_type=jax.ShapeDtypeStruct((4, NH), jnp.int32),
    mesh=plsc.VectorSubcoreMesh(core_axis_name="c", subcore_axis_name="s"),
    scratch_types=[
        pltpu.VMEM((GROUPS, 128), jnp.int32),
        pltpu.VMEM((GROUPS, 128), jnp.int32),
        pltpu.VMEM((128,), jnp.int32),
        pltpu.VMEM((SLICE,), jnp.int32),
        pltpu.VMEM_SHARED((NH,), jnp.int32),
        pltpu.VMEM_SHARED((NH,), jnp.int32),
        pltpu.SemaphoreType.DMA,
    ],
)(_hist_body)


BN = 2048


def _enc_body(h_ref, x_ref, zin_ref, zout_ref, o_ref):
    h = h_ref[...]
    deg_in = jnp.minimum(h[0] + h[1], MAX_DEG - 1)
    deg_out = jnp.minimum(h[2] + h[3], MAX_DEG - 1)
    iota = lax.broadcasted_iota(jnp.int32, (BN, MAX_DEG), 1)
    oh_in = (deg_in[:, None] == iota).astype(jnp.float32)
    oh_out = (deg_out[:, None] == iota).astype(jnp.float32)
    o_ref[...] = (x_ref[...]
                  + jnp.dot(oh_in, zin_ref[...], preferred_element_type=jnp.float32)
                  + jnp.dot(oh_out, zout_ref[...], preferred_element_type=jnp.float32))


def kernel(x, edge_index, z_in, z_out):
    e = edge_index.astype(jnp.int32)
    e = jnp.pad(e, ((0, 0), (0, E_PAD - N_EDGES)), constant_values=PAD_ID)
    e = e.reshape(2, E_PAD // 128, 128)

    h4 = _hist(e)

    return pl.pallas_call(
        _enc_body,
        grid=(pl.cdiv(N_NODES, BN),),
        in_specs=[
            pl.BlockSpec((4, BN), lambda i: (0, i)),
            pl.BlockSpec((BN, NODE_DIM), lambda i: (i, 0)),
            pl.BlockSpec((MAX_DEG, NODE_DIM), lambda i: (0, 0)),
            pl.BlockSpec((MAX_DEG, NODE_DIM), lambda i: (0, 0)),
        ],
        out_specs=pl.BlockSpec((BN, NODE_DIM), lambda i: (i, 0)),
        out_shape=jax.ShapeDtypeStruct((N_NODES, NODE_DIM), jnp.float32),
    )(h4, x, z_in, z_out)
